# Initial kernel scaffold; baseline (speedup 1.0000x reference)
import jax
import jax.numpy as jnp
from jax.experimental import pallas as pl


def kernel(x, W1, b1, W2, b2):
    raise NotImplementedError("write your pallas kernel here")


# speedup vs baseline: 1.5268x; 1.5268x over previous
import jax
import jax.numpy as jnp
from jax.experimental import pallas as pl

_DATA = 128
_HID = 6
_NM = 4
_TB = 2048


def _fused(x_ref, w1_ref, b1_ref, m2_ref, o_ref):
    x = x_ref[...]
    cond = x[:, _DATA:_DATA + 1]
    t = x[:, _DATA + 1:_DATA + 2]
    r = jnp.where(
        cond == 0.0,
        0,
        jnp.where(cond == 1.0, jnp.where(t < 0.5, 1, 2), 3),
    )

    h = jnp.tanh(
        jax.lax.dot_general(x, w1_ref[...], (((1,), (0,)), ((), ())),
                            preferred_element_type=jnp.float32)
        + b1_ref[...]
    )

    a = jnp.concatenate([h, jnp.ones((h.shape[0], 8), jnp.float32)], axis=1)
    col = jax.lax.broadcasted_iota(jnp.int32, a.shape, 1)
    expert = jnp.where(col < _NM * _HID, col // _HID, col - _NM * _HID)
    a = a * (expert == r).astype(jnp.float32)

    o_ref[...] = jax.lax.dot_general(a, m2_ref[...], (((1,), (0,)), ((), ())),
                                     preferred_element_type=jnp.float32)


def kernel(x, W1, b1, W2, b2):
    B = x.shape[0]
    w1t = jnp.transpose(W1, (1, 0, 2)).reshape(_DATA + 1, _NM * _HID)
    w1p = jnp.concatenate(
        [w1t[:_DATA], jnp.zeros((1, _NM * _HID), jnp.float32), w1t[_DATA:]],
        axis=0)
    b1c = b1.reshape(1, _NM * _HID)
    m2 = jnp.concatenate(
        [W2.reshape(_NM * _HID, _DATA), b2,
         jnp.zeros((_NM, _DATA), jnp.float32)], axis=0)

    grid = (B // _TB,)
    return pl.pallas_call(
        _fused,
        grid=grid,
        in_specs=[
            pl.BlockSpec((_TB, _DATA + 2), lambda i: (i, 0)),
            pl.BlockSpec((_DATA + 2, _NM * _HID), lambda i: (0, 0)),
            pl.BlockSpec((1, _NM * _HID), lambda i: (0, 0)),
            pl.BlockSpec((_NM * _HID + 2 * _NM, _DATA), lambda i: (0, 0)),
        ],
        out_specs=pl.BlockSpec((_TB, _DATA), lambda i: (i, 0)),
        out_shape=jax.ShapeDtypeStruct((B, _DATA), jnp.float32),
    )(x, w1p, b1c, m2)


# speedup vs baseline: 1.5306x; 1.0025x over previous
import jax
import jax.numpy as jnp
from jax.experimental import pallas as pl
from jax.experimental.pallas import tpu as pltpu

_DATA = 128
_HID = 6
_NM = 4
_TB = 2048


def _fused(x_ref, w1_ref, b1_ref, m2_ref, o_ref):
    x = x_ref[...]
    cond = x[:, _DATA:_DATA + 1]
    t = x[:, _DATA + 1:_DATA + 2]
    r = jnp.where(
        cond == 0.0,
        0,
        jnp.where(cond == 1.0, jnp.where(t < 0.5, 1, 2), 3),
    )

    h = jnp.tanh(
        jax.lax.dot_general(x, w1_ref[...], (((1,), (0,)), ((), ())),
                            preferred_element_type=jnp.float32)
        + b1_ref[...]
    )

    a = jnp.concatenate([h, jnp.ones((h.shape[0], 8), jnp.float32)], axis=1)
    col = jax.lax.broadcasted_iota(jnp.int32, a.shape, 1)
    expert = jnp.where(col < _NM * _HID, col // _HID, col - _NM * _HID)
    a = a * (expert == r).astype(jnp.float32)

    o_ref[...] = jax.lax.dot_general(a, m2_ref[...], (((1,), (0,)), ((), ())),
                                     preferred_element_type=jnp.float32)


def kernel(x, W1, b1, W2, b2):
    B = x.shape[0]
    w1t = jnp.transpose(W1, (1, 0, 2)).reshape(_DATA + 1, _NM * _HID)
    w1p = jnp.concatenate(
        [w1t[:_DATA], jnp.zeros((1, _NM * _HID), jnp.float32), w1t[_DATA:]],
        axis=0)
    b1c = b1.reshape(1, _NM * _HID)
    m2 = jnp.concatenate(
        [W2.reshape(_NM * _HID, _DATA), b2,
         jnp.zeros((_NM, _DATA), jnp.float32)], axis=0)

    grid = (B // _TB,)
    return pl.pallas_call(
        _fused,
        grid=grid,
        in_specs=[
            pl.BlockSpec((_TB, _DATA + 2), lambda i: (i, 0)),
            pl.BlockSpec((_DATA + 2, _NM * _HID), lambda i: (0, 0)),
            pl.BlockSpec((1, _NM * _HID), lambda i: (0, 0)),
            pl.BlockSpec((_NM * _HID + 2 * _NM, _DATA), lambda i: (0, 0)),
        ],
        out_specs=pl.BlockSpec((_TB, _DATA), lambda i: (i, 0)),
        out_shape=jax.ShapeDtypeStruct((B, _DATA), jnp.float32),
        compiler_params=pltpu.CompilerParams(
            dimension_semantics=("parallel",)),
    )(x, w1p, b1c, m2)


# speedup vs baseline: 1.6943x; 1.1070x over previous
import jax
import jax.numpy as jnp
from jax.experimental import pallas as pl
from jax.experimental.pallas import tpu as pltpu

_DATA = 128
_HID = 6
_NM = 4
_TB = 8192


def _fused(x_ref, w1_ref, b1_ref, m2_ref, o_ref):
    x = x_ref[...]
    cond = x[:, _DATA:_DATA + 1]
    t = x[:, _DATA + 1:_DATA + 2]
    r = jnp.where(
        cond == 0.0,
        0,
        jnp.where(cond == 1.0, jnp.where(t < 0.5, 1, 2), 3),
    )

    h = jnp.tanh(
        jax.lax.dot_general(x, w1_ref[...], (((1,), (0,)), ((), ())),
                            preferred_element_type=jnp.float32)
        + b1_ref[...]
    )

    a = jnp.concatenate([h, jnp.ones((h.shape[0], 8), jnp.float32)], axis=1)
    col = jax.lax.broadcasted_iota(jnp.int32, a.shape, 1)
    expert = jnp.where(col < _NM * _HID, col // _HID, col - _NM * _HID)
    a = a * (expert == r).astype(jnp.float32)

    o_ref[...] = jax.lax.dot_general(a, m2_ref[...], (((1,), (0,)), ((), ())),
                                     preferred_element_type=jnp.float32)


def kernel(x, W1, b1, W2, b2):
    B = x.shape[0]
    w1t = jnp.transpose(W1, (1, 0, 2)).reshape(_DATA + 1, _NM * _HID)
    w1p = jnp.concatenate(
        [w1t[:_DATA], jnp.zeros((1, _NM * _HID), jnp.float32), w1t[_DATA:]],
        axis=0)
    b1c = b1.reshape(1, _NM * _HID)
    m2 = jnp.concatenate(
        [W2.reshape(_NM * _HID, _DATA), b2,
         jnp.zeros((_NM, _DATA), jnp.float32)], axis=0)

    grid = (B // _TB,)
    return pl.pallas_call(
        _fused,
        grid=grid,
        in_specs=[
            pl.BlockSpec((_TB, _DATA + 2), lambda i: (i, 0)),
            pl.BlockSpec((_DATA + 2, _NM * _HID), lambda i: (0, 0)),
            pl.BlockSpec((1, _NM * _HID), lambda i: (0, 0)),
            pl.BlockSpec((_NM * _HID + 2 * _NM, _DATA), lambda i: (0, 0)),
        ],
        out_specs=pl.BlockSpec((_TB, _DATA), lambda i: (i, 0)),
        out_shape=jax.ShapeDtypeStruct((B, _DATA), jnp.float32),
        compiler_params=pltpu.CompilerParams(
            dimension_semantics=("parallel",)),
    )(x, w1p, b1c, m2)
